# baseline (speedup 1.0000x reference)
import numpy as np
import jax
import jax.numpy as jnp
from jax.experimental import pallas as pl
from jax.experimental.pallas import tpu as pltpu

_NK = 11
_SIGMA = 0.1
_EXACT_SIGMA = 0.001
_LOG2E = 1.4426950408889634
_LQ = 32
_LD = 1024
_EMB = 128
_NF = 128
_CH = 256


def _bin_constants():
    mus, stds = [], []
    for i in range(_NK):
        mu = 1.0 / (_NK - 1) + 2.0 * i / (_NK - 1) - 1.0
        if mu > 1:
            mus.append(1.0)
            stds.append(_EXACT_SIGMA)
        else:
            mus.append(mu)
            stds.append(_SIGMA)
    return mus, stds


_MUS, _STDS = _bin_constants()
_CSCALE = [float(np.float32(-0.5 * _LOG2E / (s * s))) for s in _STDS]
_MUS32 = [float(np.float32(m)) for m in _MUS]


def _shift_cat(x, zrow):
    a1 = jnp.concatenate([x[1:], zrow], axis=0)
    a2 = jnp.concatenate([x[2:], zrow, zrow], axis=0)
    return jnp.concatenate([x, a1, a2], axis=1)


def _knrm_body(qe_ref, de_ref, w_ref, b_ref, lw_ref, out_ref, dn_scr):
    f32 = jnp.float32
    qe = qe_ref[0]
    de = de_ref[0]

    dss = jnp.sum(de * de, axis=1, keepdims=True)
    dn = de * jax.lax.rsqrt(dss)
    qss = jnp.sum(qe * qe, axis=1, keepdims=True)
    qn = qe * jax.lax.rsqrt(qss)

    dn_scr[0:_LD, :] = dn
    dn_scr[_LD:_LD + 8, :] = jnp.zeros((8, _EMB), f32)

    w = w_ref[...]
    b = b_ref[...]
    zrow = jnp.zeros((1, _EMB), f32)

    qcat = _shift_cat(qn, zrow)
    qz = jnp.dot(qcat, w, preferred_element_type=f32, precision=jax.lax.Precision.HIGHEST) + b
    qc = jnp.tanh(qz)

    contract = (((1,), (1,)), ((), ()))
    acc = jnp.zeros((_LQ, 3 * _NK), f32)

    for c in range(_LD // _CH):
        x = dn_scr[pl.ds(c * _CH, _CH + 8), :]
        dcat = jnp.concatenate(
            [x[0:_CH], x[1:_CH + 1], x[2:_CH + 2]], axis=1)
        dz = jnp.dot(dcat, w, preferred_element_type=f32, precision=jax.lax.Precision.HIGHEST) + b
        dc = jnp.tanh(dz)

        parts = []
        for n in range(3):
            qcn = qc[:, n * _NF:(n + 1) * _NF]
            dcn = dc[:, n * _NF:(n + 1) * _NF]
            sim = jax.lax.dot_general(
                qcn, dcn, contract, preferred_element_type=f32, precision=jax.lax.Precision.HIGHEST)
            for k in range(_NK):
                t = sim - _MUS32[k]
                p = jnp.exp2((t * t) * _CSCALE[k])
                parts.append(jnp.sum(p, axis=1, keepdims=True))
        acc = acc + jnp.concatenate(parts, axis=1)

    lg = jnp.log1p(acc)
    out_ref[0] = jnp.sum(lg * lw_ref[...], keepdims=True)


def _build_call(B, interpret=False):
    grid = (B,)
    return pl.pallas_call(
        _knrm_body,
        grid=grid,
        in_specs=[
            pl.BlockSpec((1, _LQ, _EMB), lambda b: (b, 0, 0)),
            pl.BlockSpec((1, _LD, _EMB), lambda b: (b, 0, 0)),
            pl.BlockSpec((3 * _EMB, 3 * _NF), lambda b: (0, 0)),
            pl.BlockSpec((1, 3 * _NF), lambda b: (0, 0)),
            pl.BlockSpec((1, 3 * _NK), lambda b: (0, 0)),
        ],
        out_specs=pl.BlockSpec((1, 1, 1), lambda b: (b, 0, 0)),
        out_shape=jax.ShapeDtypeStruct((B, 1, 1), jnp.float32),
        scratch_shapes=[pltpu.VMEM((_LD + 8, _EMB), jnp.float32)],
        compiler_params=pltpu.CompilerParams(
            dimension_semantics=("parallel",),
        ),
        interpret=interpret,
    )


def kernel(q, d, qlen, dlen, emb_table,
           conv_w1, conv_b1, conv_w2, conv_b2, conv_w3, conv_b3, linear_w):
    B = q.shape[0]
    f32 = jnp.float32

    q_emb = jnp.take(emb_table, q[:, :, 0], axis=0).astype(f32)
    d_emb = jnp.take(emb_table, d[:, :, 0], axis=0).astype(f32)

    z = jnp.zeros((_EMB, _NF), f32)
    w1t = jnp.transpose(conv_w1[:, :, 0])
    w2t0 = jnp.transpose(conv_w2[:, :, 0])
    w2t1 = jnp.transpose(conv_w2[:, :, 1])
    w3t0 = jnp.transpose(conv_w3[:, :, 0])
    w3t1 = jnp.transpose(conv_w3[:, :, 1])
    w3t2 = jnp.transpose(conv_w3[:, :, 2])
    wbig = jnp.concatenate([
        jnp.concatenate([w1t, w2t0, w3t0], axis=1),
        jnp.concatenate([z, w2t1, w3t1], axis=1),
        jnp.concatenate([z, z, w3t2], axis=1),
    ], axis=0)
    bias = jnp.concatenate([conv_b1, conv_b2, conv_b3])[None, :]
    lw = linear_w.astype(f32)

    out = _build_call(B)(q_emb, d_emb, wbig, bias, lw)
    return out[:, 0, 0]
